# Initial kernel scaffold; baseline (speedup 1.0000x reference)
import jax
import jax.numpy as jnp
from jax.experimental import pallas as pl


def kernel(x, up, gate, down, router, up_s, gate_s, down_s):
    raise NotImplementedError("write your pallas kernel here")


# speedup vs baseline: 134.0213x; 134.0213x over previous
import functools

import jax
import jax.numpy as jnp
from jax import lax
from jax.experimental import pallas as pl
from jax.experimental.pallas import tpu as pltpu

TM1 = 128
TM = 128
KH = 2


def _dot(a, b):
    return lax.dot_general(a, b, (((1,), (1,)), ((), ())),
                           preferred_element_type=jnp.float32)


def _silu(v):
    return v * (1.0 / (1.0 + jnp.exp(-v)))


def _k1(x_ref, rt_ref, us_ref, gs_ref, ds_ref, idx_ref, ys_ref):
    xt = x_ref[...]
    logits = _dot(xt, rt_ref[...])
    mx = jnp.max(logits, axis=1, keepdims=True)
    col = lax.broadcasted_iota(jnp.int32, logits.shape, 1)
    idx = jnp.min(jnp.where(logits == mx, col, jnp.int32(2**30)), axis=1)
    idx_ref[0, 0, :] = idx.astype(jnp.int32)
    u = _dot(xt, us_ref[...])
    g = _dot(xt, gs_ref[...])
    h = _silu(g) * u
    ys_ref[...] = _dot(h, ds_ref[...])


def _k2(g_r, t_r, rs_r, re_r, fi_r, perm_r,
        x_ref, ys_ref, up_ref, gate_ref, down_ref,
        o_ref, x_scr, y_scr):
    w = pl.program_id(0)
    k = pl.program_id(1)
    t = t_r[w]
    rs = rs_r[w]
    re = re_r[w]

    @pl.when((fi_r[w] == 1) & (k == 0))
    def _gather():
        def body(r, _):
            p = perm_r[t * TM + r]
            x_scr[pl.ds(r, 1), :] = x_ref[pl.ds(p, 1), :]
            return 0
        lax.fori_loop(0, TM, body, 0)

    xt = x_scr[...]
    u = _dot(xt, up_ref[0])
    g = _dot(xt, gate_ref[0])
    h = _silu(g) * u
    contrib = _dot(h, down_ref[0])

    @pl.when(k == 0)
    def _init():
        y_scr[...] = contrib

    @pl.when(k != 0)
    def _acc():
        y_scr[...] += contrib

    @pl.when(k == KH - 1)
    def _scatter():
        def scat(r, _):
            p = perm_r[t * TM + r]
            o_ref[pl.ds(p, 1), :] = (y_scr[pl.ds(r, 1), :]
                                     + ys_ref[pl.ds(p, 1), :])
            return 0
        lax.fori_loop(rs, re, scat, 0)


def _schedule(idx, E, BT):
    ntiles = BT // TM
    W = ntiles + E - 1
    perm = jnp.argsort(idx).astype(jnp.int32)
    sizes = jnp.sum(idx[None, :] == jnp.arange(E, dtype=idx.dtype)[:, None],
                    axis=1).astype(jnp.int32)
    ends = jnp.cumsum(sizes)
    starts = ends - sizes
    t0 = starts // TM
    t1 = (ends + TM - 1) // TM
    u = jnp.where(sizes > 0, t1 - t0, 0)
    uend = jnp.cumsum(u)
    ustart = uend - u
    U = uend[E - 1]
    w = jnp.arange(W, dtype=jnp.int32)
    gw = jnp.clip(jnp.searchsorted(uend, w, side="right"), 0, E - 1)
    gw = gw.astype(jnp.int32)
    tile = t0[gw] + (w - ustart[gw])
    rs = jnp.maximum(starts[gw] - tile * TM, 0)
    re = jnp.minimum(ends[gw] - tile * TM, TM)
    valid = w < U
    last_g = jnp.max(jnp.where(sizes > 0, jnp.arange(E, dtype=jnp.int32), -1))
    gw = jnp.where(valid, gw, last_g).astype(jnp.int32)
    tile = jnp.where(valid, tile, ntiles - 1).astype(jnp.int32)
    rs = jnp.where(valid, rs, 0).astype(jnp.int32)
    re = jnp.where(valid, re, 0).astype(jnp.int32)
    first = jnp.concatenate(
        [jnp.ones((1,), jnp.int32),
         (tile[1:] != tile[:-1]).astype(jnp.int32)])
    return gw, tile, rs, re, first, perm


def kernel(x, up, gate, down, router, up_s, gate_s, down_s):
    b, t, c = x.shape
    BT = b * t
    E, H, C = up.shape
    x2 = x.reshape(BT, c)

    idx3, ys = pl.pallas_call(
        _k1,
        grid=(BT // TM1,),
        in_specs=[
            pl.BlockSpec((TM1, C), lambda i: (i, 0)),
            pl.BlockSpec((E, C), lambda i: (0, 0)),
            pl.BlockSpec((H, C), lambda i: (0, 0)),
            pl.BlockSpec((H, C), lambda i: (0, 0)),
            pl.BlockSpec((C, H), lambda i: (0, 0)),
        ],
        out_specs=[
            pl.BlockSpec((1, 1, TM1), lambda i: (i, 0, 0)),
            pl.BlockSpec((TM1, C), lambda i: (i, 0)),
        ],
        out_shape=[
            jax.ShapeDtypeStruct((BT // TM1, 1, TM1), jnp.int32),
            jax.ShapeDtypeStruct((BT, C), jnp.float32),
        ],
    )(x2, router, up_s, gate_s, down_s)
    idx = idx3.reshape(BT)

    gw, tile, rs, re, first, perm = _schedule(idx, E, BT)
    W = BT // TM + E - 1

    y2 = pl.pallas_call(
        _k2,
        grid_spec=pltpu.PrefetchScalarGridSpec(
            num_scalar_prefetch=6,
            grid=(W, KH),
            in_specs=[
                pl.BlockSpec((BT, C), lambda w, k, *s: (0, 0)),
                pl.BlockSpec((BT, C), lambda w, k, *s: (0, 0)),
                pl.BlockSpec((1, H // KH, C),
                             lambda w, k, g, t_, r1, r2, f, p: (g[w], k, 0)),
                pl.BlockSpec((1, H // KH, C),
                             lambda w, k, g, t_, r1, r2, f, p: (g[w], k, 0)),
                pl.BlockSpec((1, C, H // KH),
                             lambda w, k, g, t_, r1, r2, f, p: (g[w], 0, k)),
            ],
            out_specs=pl.BlockSpec((BT, C), lambda w, k, *s: (0, 0)),
            scratch_shapes=[
                pltpu.VMEM((TM, C), jnp.float32),
                pltpu.VMEM((TM, C), jnp.float32),
            ],
        ),
        out_shape=jax.ShapeDtypeStruct((BT, C), jnp.float32),
    )(gw, tile, rs, re, first, perm, x2, ys, up, gate, down)

    return y2.reshape(b, t, c)


# speedup vs baseline: 144.2675x; 1.0765x over previous
import functools

import jax
import jax.numpy as jnp
from jax import lax
from jax.experimental import pallas as pl
from jax.experimental.pallas import tpu as pltpu
from jax.experimental.pallas import tpu_sc as plsc

TM1 = 128
TM = 128


def _dot(a, b):
    return lax.dot_general(a, b, (((1,), (1,)), ((), ())),
                           preferred_element_type=jnp.float32)


def _silu(v):
    return v * (1.0 / (1.0 + jnp.exp(-v)))


def _k1(x_ref, rt_ref, us_ref, gs_ref, ds_ref, idx_ref, ys_ref):
    xt = x_ref[...]
    logits = _dot(xt, rt_ref[...])
    mx = jnp.max(logits, axis=1, keepdims=True)
    col = lax.broadcasted_iota(jnp.int32, logits.shape, 1)
    idx = jnp.min(jnp.where(logits == mx, col, jnp.int32(2**30)), axis=1)
    idx_ref[0, 0, :] = idx.astype(jnp.int32)
    u = _dot(xt, us_ref[...])
    g = _dot(xt, gs_ref[...])
    h = _silu(g) * u
    ys_ref[...] = _dot(h, ds_ref[...])


def _k2(g_r, t_r, rs_r, re_r,
        xs_ref, yss_ref, up_ref, gate_ref, down_ref, o_ref):
    w = pl.program_id(0)
    rs = rs_r[w]
    re = re_r[w]
    xt = xs_ref[...]
    u = _dot(xt, up_ref[0])
    g = _dot(xt, gate_ref[0])
    h = _silu(g) * u
    val = _dot(h, down_ref[0]) + yss_ref[...]
    rows = lax.broadcasted_iota(jnp.int32, (TM, 1), 0)
    mask = (rows >= rs) & (rows < re)
    o_ref[...] = jnp.where(mask, val, o_ref[...])


def _schedule(idx, E, BT):
    ntiles = BT // TM
    W = ntiles + E - 1
    perm = jnp.argsort(idx).astype(jnp.int32)
    sizes = jnp.sum(idx[None, :] == jnp.arange(E, dtype=idx.dtype)[:, None],
                    axis=1).astype(jnp.int32)
    ends = jnp.cumsum(sizes)
    starts = ends - sizes
    t0 = starts // TM
    t1 = (ends + TM - 1) // TM
    u = jnp.where(sizes > 0, t1 - t0, 0)
    uend = jnp.cumsum(u)
    ustart = uend - u
    U = uend[E - 1]
    w = jnp.arange(W, dtype=jnp.int32)
    gw = jnp.clip(jnp.searchsorted(uend, w, side="right"), 0, E - 1)
    gw = gw.astype(jnp.int32)
    tile = t0[gw] + (w - ustart[gw])
    rs = jnp.maximum(starts[gw] - tile * TM, 0)
    re = jnp.minimum(ends[gw] - tile * TM, TM)
    valid = w < U
    last_g = jnp.max(jnp.where(sizes > 0, jnp.arange(E, dtype=jnp.int32), -1))
    gw = jnp.where(valid, gw, last_g).astype(jnp.int32)
    tile = jnp.where(valid, tile, ntiles - 1).astype(jnp.int32)
    rs = jnp.where(valid, rs, 0).astype(jnp.int32)
    re = jnp.where(valid, re, 0).astype(jnp.int32)
    return gw, tile, rs, re, perm


_SC_MESH = plsc.VectorSubcoreMesh(core_axis_name="c", subcore_axis_name="s")
_NW = 32
_RPW = 2048 // _NW


def _sc_gather_body(x_hbm, ys_hbm, perm_hbm, xs_out, yss_out,
                    idx_v, rows_v, sem):
    wid = lax.axis_index("s") * 2 + lax.axis_index("c")
    base = wid * _RPW
    pltpu.sync_copy(perm_hbm.at[pl.ds(base, _RPW)], idx_v)
    pltpu.async_copy(x_hbm.at[idx_v], rows_v, sem).wait()
    pltpu.sync_copy(rows_v, xs_out.at[pl.ds(base, _RPW)])
    pltpu.async_copy(ys_hbm.at[idx_v], rows_v, sem).wait()
    pltpu.sync_copy(rows_v, yss_out.at[pl.ds(base, _RPW)])


def _sc_scatter_body(ysort_hbm, perm_hbm, y_out, idx_v, rows_v, sem):
    wid = lax.axis_index("s") * 2 + lax.axis_index("c")
    base = wid * _RPW
    pltpu.sync_copy(perm_hbm.at[pl.ds(base, _RPW)], idx_v)
    pltpu.sync_copy(ysort_hbm.at[pl.ds(base, _RPW)], rows_v)
    pltpu.async_copy(rows_v, y_out.at[idx_v], sem).wait()


def kernel(x, up, gate, down, router, up_s, gate_s, down_s):
    b, t, c = x.shape
    BT = b * t
    E, H, C = up.shape
    x2 = x.reshape(BT, c)

    idx3, ys = pl.pallas_call(
        _k1,
        grid=(BT // TM1,),
        in_specs=[
            pl.BlockSpec((TM1, C), lambda i: (i, 0)),
            pl.BlockSpec((E, C), lambda i: (0, 0)),
            pl.BlockSpec((H, C), lambda i: (0, 0)),
            pl.BlockSpec((H, C), lambda i: (0, 0)),
            pl.BlockSpec((C, H), lambda i: (0, 0)),
        ],
        out_specs=[
            pl.BlockSpec((1, 1, TM1), lambda i: (i, 0, 0)),
            pl.BlockSpec((TM1, C), lambda i: (i, 0)),
        ],
        out_shape=[
            jax.ShapeDtypeStruct((BT // TM1, 1, TM1), jnp.int32),
            jax.ShapeDtypeStruct((BT, C), jnp.float32),
        ],
    )(x2, router, up_s, gate_s, down_s)
    idx = idx3.reshape(BT)

    gw, tile, rs, re, perm = _schedule(idx, E, BT)
    W = BT // TM + E - 1

    sc_gather = functools.partial(
        pl.kernel,
        mesh=_SC_MESH,
        out_type=[
            jax.ShapeDtypeStruct((BT, C), jnp.float32),
            jax.ShapeDtypeStruct((BT, C), jnp.float32),
        ],
        scratch_types=[
            pltpu.VMEM((_RPW,), jnp.int32),
            pltpu.VMEM((_RPW, C), jnp.float32),
            pltpu.SemaphoreType.DMA,
        ],
    )(_sc_gather_body)
    xs, yss = sc_gather(x2, ys, perm)

    y_sorted = pl.pallas_call(
        _k2,
        grid_spec=pltpu.PrefetchScalarGridSpec(
            num_scalar_prefetch=4,
            grid=(W,),
            in_specs=[
                pl.BlockSpec((TM, C), lambda w, g, t_, r1, r2: (t_[w], 0)),
                pl.BlockSpec((TM, C), lambda w, g, t_, r1, r2: (t_[w], 0)),
                pl.BlockSpec((1, H, C), lambda w, g, t_, r1, r2: (g[w], 0, 0)),
                pl.BlockSpec((1, H, C), lambda w, g, t_, r1, r2: (g[w], 0, 0)),
                pl.BlockSpec((1, C, H), lambda w, g, t_, r1, r2: (g[w], 0, 0)),
            ],
            out_specs=pl.BlockSpec((TM, C), lambda w, g, t_, r1, r2: (t_[w], 0)),
        ),
        out_shape=jax.ShapeDtypeStruct((BT, C), jnp.float32),
    )(gw, tile, rs, re, xs, yss, up, gate, down)

    sc_scatter = functools.partial(
        pl.kernel,
        mesh=_SC_MESH,
        out_type=jax.ShapeDtypeStruct((BT, C), jnp.float32),
        scratch_types=[
            pltpu.VMEM((_RPW,), jnp.int32),
            pltpu.VMEM((_RPW, C), jnp.float32),
            pltpu.SemaphoreType.DMA,
        ],
    )(_sc_scatter_body)
    y2 = sc_scatter(y_sorted, perm)

    return y2.reshape(b, t, c)


# speedup vs baseline: 144.6175x; 1.0024x over previous
import functools

import jax
import jax.numpy as jnp
from jax import lax
from jax.experimental import pallas as pl
from jax.experimental.pallas import tpu as pltpu
from jax.experimental.pallas import tpu_sc as plsc

TM1 = 128
TM = 128


def _dot(a, b):
    return lax.dot_general(a, b, (((1,), (1,)), ((), ())),
                           preferred_element_type=jnp.float32)


def _dot_fast(a, b):
    return lax.dot_general(a, b, (((1,), (1,)), ((), ())),
                           preferred_element_type=jnp.float32,
                           precision=lax.Precision.DEFAULT)


def _silu(v):
    return v * (1.0 / (1.0 + jnp.exp(-v)))


def _k1(x_ref, rt_ref, us_ref, gs_ref, ds_ref, idx_ref, ys_ref):
    xt = x_ref[...]
    logits = _dot(xt, rt_ref[...])
    mx = jnp.max(logits, axis=1, keepdims=True)
    col = lax.broadcasted_iota(jnp.int32, logits.shape, 1)
    idx = jnp.min(jnp.where(logits == mx, col, jnp.int32(2**30)), axis=1)
    idx_ref[0, 0, :] = idx.astype(jnp.int32)
    u = _dot_fast(xt, us_ref[...])
    g = _dot_fast(xt, gs_ref[...])
    h = _silu(g) * u
    ys_ref[...] = _dot_fast(h, ds_ref[...])


def _k2(g_r, t_r, rs_r, re_r,
        xs_ref, yss_ref, up_ref, gate_ref, down_ref, o_ref):
    w = pl.program_id(0)
    rs = rs_r[w]
    re = re_r[w]
    xt = xs_ref[...]
    u = _dot_fast(xt, up_ref[0])
    g = _dot_fast(xt, gate_ref[0])
    h = _silu(g) * u
    val = _dot_fast(h, down_ref[0]) + yss_ref[...]
    rows = lax.broadcasted_iota(jnp.int32, (TM, 1), 0)
    mask = (rows >= rs) & (rows < re)
    o_ref[...] = jnp.where(mask, val, o_ref[...])


def _schedule(idx, E, BT):
    ntiles = BT // TM
    W = ntiles + E - 1
    perm = jnp.argsort(idx).astype(jnp.int32)
    sizes = jnp.sum(idx[None, :] == jnp.arange(E, dtype=idx.dtype)[:, None],
                    axis=1).astype(jnp.int32)
    ends = jnp.cumsum(sizes)
    starts = ends - sizes
    t0 = starts // TM
    t1 = (ends + TM - 1) // TM
    u = jnp.where(sizes > 0, t1 - t0, 0)
    uend = jnp.cumsum(u)
    ustart = uend - u
    U = uend[E - 1]
    w = jnp.arange(W, dtype=jnp.int32)
    gw = jnp.clip(jnp.searchsorted(uend, w, side="right"), 0, E - 1)
    gw = gw.astype(jnp.int32)
    tile = t0[gw] + (w - ustart[gw])
    rs = jnp.maximum(starts[gw] - tile * TM, 0)
    re = jnp.minimum(ends[gw] - tile * TM, TM)
    valid = w < U
    last_g = jnp.max(jnp.where(sizes > 0, jnp.arange(E, dtype=jnp.int32), -1))
    gw = jnp.where(valid, gw, last_g).astype(jnp.int32)
    tile = jnp.where(valid, tile, ntiles - 1).astype(jnp.int32)
    rs = jnp.where(valid, rs, 0).astype(jnp.int32)
    re = jnp.where(valid, re, 0).astype(jnp.int32)
    return gw, tile, rs, re, perm


_NW = 32
_RPW = 2048 // _NW


def _sc_gather_body(x_hbm, ys_hbm, perm_hbm, xs_out, yss_out,
                    idx_v, rows_v, sem):
    wid = lax.axis_index("s") * 2 + lax.axis_index("c")
    base = wid * _RPW
    pltpu.sync_copy(perm_hbm.at[pl.ds(base, _RPW)], idx_v)
    pltpu.async_copy(x_hbm.at[idx_v], rows_v, sem).wait()
    pltpu.sync_copy(rows_v, xs_out.at[pl.ds(base, _RPW)])
    pltpu.async_copy(ys_hbm.at[idx_v], rows_v, sem).wait()
    pltpu.sync_copy(rows_v, yss_out.at[pl.ds(base, _RPW)])


def _sc_scatter_body(ysort_hbm, perm_hbm, y_out, idx_v, rows_v, sem):
    wid = lax.axis_index("s") * 2 + lax.axis_index("c")
    base = wid * _RPW
    pltpu.sync_copy(perm_hbm.at[pl.ds(base, _RPW)], idx_v)
    pltpu.sync_copy(ysort_hbm.at[pl.ds(base, _RPW)], rows_v)
    pltpu.async_copy(rows_v, y_out.at[idx_v], sem).wait()


def kernel(x, up, gate, down, router, up_s, gate_s, down_s):
    b, t, c = x.shape
    BT = b * t
    E, H, C = up.shape
    x2 = x.reshape(BT, c)

    idx3, ys = pl.pallas_call(
        _k1,
        grid=(BT // TM1,),
        in_specs=[
            pl.BlockSpec((TM1, C), lambda i: (i, 0)),
            pl.BlockSpec((E, C), lambda i: (0, 0)),
            pl.BlockSpec((H, C), lambda i: (0, 0)),
            pl.BlockSpec((H, C), lambda i: (0, 0)),
            pl.BlockSpec((C, H), lambda i: (0, 0)),
        ],
        out_specs=[
            pl.BlockSpec((1, 1, TM1), lambda i: (i, 0, 0)),
            pl.BlockSpec((TM1, C), lambda i: (i, 0)),
        ],
        out_shape=[
            jax.ShapeDtypeStruct((BT // TM1, 1, TM1), jnp.int32),
            jax.ShapeDtypeStruct((BT, C), jnp.float32),
        ],
    )(x2, router, up_s, gate_s, down_s)
    idx = idx3.reshape(BT)

    gw, tile, rs, re, perm = _schedule(idx, E, BT)
    W = BT // TM + E - 1

    sc_mesh = plsc.VectorSubcoreMesh(core_axis_name="c", subcore_axis_name="s")
    sc_gather = functools.partial(
        pl.kernel,
        mesh=sc_mesh,
        out_type=[
            jax.ShapeDtypeStruct((BT, C), jnp.float32),
            jax.ShapeDtypeStruct((BT, C), jnp.float32),
        ],
        scratch_types=[
            pltpu.VMEM((_RPW,), jnp.int32),
            pltpu.VMEM((_RPW, C), jnp.float32),
            pltpu.SemaphoreType.DMA,
        ],
    )(_sc_gather_body)
    xs, yss = sc_gather(x2, ys, perm)

    y_sorted = pl.pallas_call(
        _k2,
        grid_spec=pltpu.PrefetchScalarGridSpec(
            num_scalar_prefetch=4,
            grid=(W,),
            in_specs=[
                pl.BlockSpec((TM, C), lambda w, g, t_, r1, r2: (t_[w], 0)),
                pl.BlockSpec((TM, C), lambda w, g, t_, r1, r2: (t_[w], 0)),
                pl.BlockSpec((1, H, C), lambda w, g, t_, r1, r2: (g[w], 0, 0)),
                pl.BlockSpec((1, H, C), lambda w, g, t_, r1, r2: (g[w], 0, 0)),
                pl.BlockSpec((1, C, H), lambda w, g, t_, r1, r2: (g[w], 0, 0)),
            ],
            out_specs=pl.BlockSpec((TM, C), lambda w, g, t_, r1, r2: (t_[w], 0)),
        ),
        out_shape=jax.ShapeDtypeStruct((BT, C), jnp.float32),
    )(gw, tile, rs, re, xs, yss, up, gate, down)

    sc_scatter = functools.partial(
        pl.kernel,
        mesh=sc_mesh,
        out_type=jax.ShapeDtypeStruct((BT, C), jnp.float32),
        scratch_types=[
            pltpu.VMEM((_RPW,), jnp.int32),
            pltpu.VMEM((_RPW, C), jnp.float32),
            pltpu.SemaphoreType.DMA,
        ],
    )(_sc_scatter_body)
    y2 = sc_scatter(y_sorted, perm)

    return y2.reshape(b, t, c)
